# Initial kernel scaffold; baseline (speedup 1.0000x reference)
import jax
import jax.numpy as jnp
from jax.experimental import pallas as pl


def kernel(emb, key_w, val_w, query, node_w, target_w, node_strings, node_keys, edge_index, edge_types):
    raise NotImplementedError("write your pallas kernel here")


# speedup vs baseline: 24.3084x; 24.3084x over previous
import functools

import jax
import jax.numpy as jnp
import numpy as np
from jax import lax
from jax.experimental import pallas as pl
from jax.experimental.pallas import tpu as pltpu
from jax.experimental.pallas import tpu_sc as plsc

N = 10000
E = 320000
D = 128
H = 8
DH = 16
NB = 2
NL = 2
NE = 8
NK = 16

HH = H // 2
HW = HH * DH

NC = 2
NSC = 16
NW = NC * NSC
EW = E // NW
C = 80
G = C // 16
NCH = EW // C
NPAD = 10240
RPT = NPAD // NSC
AW = HW + 16

NP = 10240
RW = NP // NW

_mesh = plsc.VectorSubcoreMesh(core_axis_name="c", subcore_axis_name="s")

_sc_params = pltpu.CompilerParams(needs_layout_passes=False,
                                  use_tc_tiling_on_sc=False)


@functools.partial(
    pl.kernel, mesh=_mesh,
    out_type=jax.ShapeDtypeStruct((NP, D), jnp.float32),
    scratch_types=[
        pltpu.VMEM((RW,), jnp.int32),
        pltpu.VMEM((RW, D), jnp.float32),
        pltpu.SemaphoreType.DMA,
    ],
    compiler_params=_sc_params,
)
def _gather_rows(table_hbm, idx_hbm, out_hbm, idxv, rows, sem):
    wid = lax.axis_index("c") * NSC + lax.axis_index("s")
    base = wid * RW
    pltpu.sync_copy(idx_hbm.at[pl.ds(base, RW)], idxv)
    pltpu.async_copy(table_hbm.at[idxv], rows, sem).wait()
    pltpu.sync_copy(rows, out_hbm.at[pl.ds(base, RW)])


@functools.partial(
    pl.kernel, mesh=_mesh,
    out_type=[
        jax.ShapeDtypeStruct((NP, HW), jnp.float32),
        jax.ShapeDtypeStruct((NP, HW), jnp.float32),
    ],
    scratch_types=[
        pltpu.VMEM((RW,), jnp.int32),
        pltpu.VMEM((RW, D), jnp.float32),
        pltpu.SemaphoreType.DMA,
    ],
    compiler_params=_sc_params,
)
def _gather_rows_split(table_hbm, idx_hbm, out_a, out_b, idxv, rows, sem):
    wid = lax.axis_index("c") * NSC + lax.axis_index("s")
    base = wid * RW
    pltpu.sync_copy(idx_hbm.at[pl.ds(base, RW)], idxv)
    pltpu.async_copy(table_hbm.at[idxv], rows, sem).wait()
    pltpu.sync_copy(rows.at[pl.ds(0, RW), pl.ds(0, HW)],
                    out_a.at[pl.ds(base, RW)])
    pltpu.sync_copy(rows.at[pl.ds(0, RW), pl.ds(HW, HW)],
                    out_b.at[pl.ds(base, RW)])


@functools.partial(
    pl.kernel, mesh=_mesh,
    out_type=[
        jax.ShapeDtypeStruct((NC, NPAD, HW), jnp.float32),
        jax.ShapeDtypeStruct((NC, NPAD, 16), jnp.float32),
    ],
    scratch_types=[
        pltpu.VMEM((EW,), jnp.int32),
        pltpu.VMEM((EW,), jnp.int32),
        pltpu.VMEM((EW,), jnp.int32),
        pltpu.VMEM((C,), jnp.int32),
        pltpu.VMEM((C,), jnp.int32),
        pltpu.VMEM((C, HW), jnp.float32),
        pltpu.VMEM((C, HW), jnp.float32),
        pltpu.VMEM((C, HW), jnp.float32),
        pltpu.VMEM((C, AW), jnp.float32),
        pltpu.VMEM_SHARED((NPAD, AW), jnp.float32),
        pltpu.SemaphoreType.DMA,
    ],
    compiler_params=_sc_params,
)
def _edge_phase(kt_hbm, vt_hbm, q_hbm, src_hbm, dst_hbm, et_hbm,
                agg_out, den_out,
                srcb, dstb, etb, kvix, dstix, kbuf, vbuf, qbuf, contrib,
                acc_sh, sem):
    c = lax.axis_index("c")
    s = lax.axis_index("s")
    ebase = (c * NSC + s) * EW
    rbase = s * RPT

    zero16 = jnp.zeros((16,), jnp.float32)

    @pl.loop(0, C)
    def _(r):
        for cc in range(AW // 16):
            contrib[r, pl.ds(cc * 16, 16)] = zero16

    @pl.loop(0, RPT // C)
    def _(j):
        pltpu.sync_copy(contrib, acc_sh.at[pl.ds(rbase + j * C, C)])
    plsc.subcore_barrier()

    pltpu.sync_copy(src_hbm.at[pl.ds(ebase, EW)], srcb)
    pltpu.sync_copy(dst_hbm.at[pl.ds(ebase, EW)], dstb)
    pltpu.sync_copy(et_hbm.at[pl.ds(ebase, EW)], etb)

    inv_sqrt_dh = float(1.0 / np.sqrt(DH))

    @pl.loop(0, NCH)
    def _(ch):
        off = ch * C
        for g in range(G):
            sl = pl.ds(off + g * 16, 16)
            kvix[pl.ds(g * 16, 16)] = etb[sl] * N + srcb[sl]
            dstix[pl.ds(g * 16, 16)] = dstb[sl]

        cp_k = pltpu.make_async_copy(kt_hbm.at[kvix], kbuf, sem)
        cp_v = pltpu.make_async_copy(vt_hbm.at[kvix], vbuf, sem)
        cp_q = pltpu.make_async_copy(q_hbm.at[dstix], qbuf, sem)
        cp_k.start()
        cp_v.start()
        cp_q.start()
        cp_k.wait()
        cp_v.wait()
        cp_q.wait()

        @pl.loop(0, G)
        def _(g):
            rows = lax.iota(jnp.int32, 16) + g * 16
            for h in range(HH):
                acc = jnp.zeros((16,), jnp.float32)
                for p in range(DH):
                    col = jnp.full((16,), h * DH + p, jnp.int32)
                    qv = plsc.load_gather(qbuf, [rows, col])
                    kv = plsc.load_gather(kbuf, [rows, col])
                    acc = acc + qv * kv
                ex = jnp.exp(acc * inv_sqrt_dh)
                plsc.store_scatter(
                    contrib, [rows, jnp.full((16,), HW + h, jnp.int32)], ex)
                for p in range(DH):
                    col = jnp.full((16,), h * DH + p, jnp.int32)
                    vv = plsc.load_gather(vbuf, [rows, col])
                    plsc.store_scatter(contrib, [rows, col], vv * ex)

        pltpu.sync_copy(contrib, acc_sh.at[dstix], add=True)

    plsc.subcore_barrier()

    @pl.loop(0, RPT // C)
    def _(j):
        r0 = rbase + j * C
        pltpu.sync_copy(acc_sh.at[pl.ds(r0, C), pl.ds(0, HW)],
                        agg_out.at[c, pl.ds(r0, C)])
        pltpu.sync_copy(acc_sh.at[pl.ds(r0, C), pl.ds(HW, 16)],
                        den_out.at[c, pl.ds(r0, C)])


TNP = 1000


def _proj_body(feat_ref, kw_ref, vw_ref, ka_ref, kb_ref, va_ref, vb_ref):
    f = feat_ref[...]
    dn = (((1,), (1,)), ((), ()))
    kt = lax.dot_general(f, kw_ref[0], dn, preferred_element_type=jnp.float32)
    vt = lax.dot_general(f, vw_ref[0], dn, preferred_element_type=jnp.float32)
    ka_ref[0] = kt[:, :HW]
    kb_ref[0] = kt[:, HW:]
    va_ref[0] = vt[:, :HW]
    vb_ref[0] = vt[:, HW:]


def _proj(feat, kwf, vwf):
    half = jax.ShapeDtypeStruct((NE, N, HW), jnp.float32)
    return pl.pallas_call(
        _proj_body,
        grid=(N // TNP, NE),
        in_specs=[
            pl.BlockSpec((TNP, D), lambda n, t: (n, 0)),
            pl.BlockSpec((1, D, D), lambda n, t: (t, 0, 0)),
            pl.BlockSpec((1, D, D), lambda n, t: (t, 0, 0)),
        ],
        out_specs=[
            pl.BlockSpec((1, TNP, HW), lambda n, t: (t, n, 0)),
            pl.BlockSpec((1, TNP, HW), lambda n, t: (t, n, 0)),
            pl.BlockSpec((1, TNP, HW), lambda n, t: (t, n, 0)),
            pl.BlockSpec((1, TNP, HW), lambda n, t: (t, n, 0)),
        ],
        out_shape=[half, half, half, half],
    )(feat, kwf, vwf)


TNO = 400


def _merge_normalize(agg_a_ref, agg_b_ref, den_a_ref, den_b_ref, nk_ref,
                     wcat_a_ref, wcat_b_ref):
    expand = (lax.broadcasted_iota(jnp.int32, (HH, HW), 0)
              == lax.broadcasted_iota(jnp.int32, (HH, HW), 1) // DH
              ).astype(jnp.float32)
    agg_a = agg_a_ref[0] + agg_a_ref[1]
    den4a = den_a_ref[0, :, :HH] + den_a_ref[1, :, :HH]
    aggna = agg_a / jnp.dot(den4a + 1e-9, expand,
                            preferred_element_type=jnp.float32)
    agg_b = agg_b_ref[0] + agg_b_ref[1]
    den4b = den_b_ref[0, :, :HH] + den_b_ref[1, :, :HH]
    aggnb = agg_b / jnp.dot(den4b + 1e-9, expand,
                            preferred_element_type=jnp.float32)
    big = (jnp.dot(aggna, wcat_a_ref[...], preferred_element_type=jnp.float32)
           + jnp.dot(aggnb, wcat_b_ref[...],
                     preferred_element_type=jnp.float32))
    nk = nk_ref[...]
    out = jnp.zeros((TNO, D), jnp.float32)
    for k in range(NK):
        m = (nk == k).astype(jnp.float32)
        out = out + m * big[:, k * D:(k + 1) * D]
    out = jnp.maximum(out, 0.0)
    mu = jnp.mean(out, axis=-1, keepdims=True)
    var = jnp.mean((out - mu) * (out - mu), axis=-1, keepdims=True)
    return (out - mu) * lax.rsqrt(var + 1e-5)


def _post_mid_body(agg_a_ref, agg_b_ref, den_a_ref, den_b_ref, nk_ref,
                   wcat_a_ref, wcat_b_ref, feat_ref):
    feat_ref[...] = _merge_normalize(agg_a_ref, agg_b_ref, den_a_ref,
                                     den_b_ref, nk_ref, wcat_a_ref, wcat_b_ref)


def _post_end_body(agg_a_ref, agg_b_ref, den_a_ref, den_b_ref, nk_ref,
                   wcat_a_ref, wcat_b_ref, prev_ref, twf_ref,
                   feat_ref, score_ref):
    f = _merge_normalize(agg_a_ref, agg_b_ref, den_a_ref, den_b_ref, nk_ref,
                         wcat_a_ref, wcat_b_ref) + prev_ref[...]
    feat_ref[...] = f
    nk = nk_ref[...]
    onehot = (nk == lax.broadcasted_iota(jnp.int32, (TNO, NK), 1)
              ).astype(jnp.float32)
    tw = jnp.dot(onehot, twf_ref[...], preferred_element_type=jnp.float32)

    @pl.when(pl.program_id(0) == 0)
    def _():
        score_ref[...] = jnp.zeros((1, 1), jnp.float32)

    score_ref[...] += (jnp.sum(f * tw) * (1.0 / N)).reshape(1, 1)


_POST_SPECS = [
    pl.BlockSpec((NC, TNO, HW), lambda n: (0, n, 0)),
    pl.BlockSpec((NC, TNO, HW), lambda n: (0, n, 0)),
    pl.BlockSpec((NC, TNO, 16), lambda n: (0, n, 0)),
    pl.BlockSpec((NC, TNO, 16), lambda n: (0, n, 0)),
    pl.BlockSpec((TNO, 1), lambda n: (n, 0)),
    pl.BlockSpec((HW, NK * D), lambda n: (0, 0)),
    pl.BlockSpec((HW, NK * D), lambda n: (0, 0)),
]


def _post_mid(agg_a, agg_b, den_a, den_b, nk2d, wcat_a, wcat_b):
    return pl.pallas_call(
        _post_mid_body,
        grid=(N // TNO,),
        in_specs=_POST_SPECS,
        out_specs=pl.BlockSpec((TNO, D), lambda n: (n, 0)),
        out_shape=jax.ShapeDtypeStruct((N, D), jnp.float32),
    )(agg_a, agg_b, den_a, den_b, nk2d, wcat_a, wcat_b)


def _post_end(agg_a, agg_b, den_a, den_b, nk2d, wcat_a, wcat_b, prev, twf):
    return pl.pallas_call(
        _post_end_body,
        grid=(N // TNO,),
        in_specs=_POST_SPECS + [
            pl.BlockSpec((TNO, D), lambda n: (n, 0)),
            pl.BlockSpec((NK, D), lambda n: (0, 0)),
        ],
        out_specs=[
            pl.BlockSpec((TNO, D), lambda n: (n, 0)),
            pl.BlockSpec((1, 1), lambda n: (0, 0)),
        ],
        out_shape=[
            jax.ShapeDtypeStruct((N, D), jnp.float32),
            jax.ShapeDtypeStruct((1, 1), jnp.float32),
        ],
    )(agg_a, agg_b, den_a, den_b, nk2d, wcat_a, wcat_b, prev, twf)


def kernel(emb, key_w, val_w, query, node_w, target_w,
           node_strings, node_keys, edge_index, edge_types):
    src = edge_index[0].astype(jnp.int32)
    dst = edge_index[1].astype(jnp.int32)
    et = edge_types.astype(jnp.int32)
    nstr = node_strings.astype(jnp.int32)
    nkey = node_keys.astype(jnp.int32)

    pad = jnp.zeros((NP - N,), jnp.int32)
    nsp = jnp.concatenate([nstr, pad])
    nkp = jnp.concatenate([nkey, pad])
    nk2d = nkey[:, None]

    feat = _gather_rows(emb, nsp)[:N]
    prev = feat
    scores = []
    for i in range(NB):
        qf = query[i].reshape(NK, D)
        q_a, q_b = _gather_rows_split(qf, nkp)
        wcat = node_w[i].reshape(NK, D, D).transpose(1, 0, 2).reshape(D, NK * D)
        wcat_a = wcat[:HW]
        wcat_b = wcat[HW:]
        twf = target_w[i].reshape(NK, D)
        kwf = key_w[i].reshape(NE, D, D)
        vwf = val_w[i].reshape(NE, D, D)
        for l in range(NL):
            ka, kb, va, vb = _proj(feat, kwf, vwf)
            agg_a, den_a = _edge_phase(ka.reshape(NE * N, HW),
                                       va.reshape(NE * N, HW),
                                       q_a, src, dst, et)
            agg_b, den_b = _edge_phase(kb.reshape(NE * N, HW),
                                       vb.reshape(NE * N, HW),
                                       q_b, src, dst, et)
            if l == NL - 1:
                feat, sc = _post_end(agg_a, agg_b, den_a, den_b, nk2d,
                                     wcat_a, wcat_b, prev, twf)
                prev = feat
                scores.append(sc[0, 0])
            else:
                feat = _post_mid(agg_a, agg_b, den_a, den_b, nk2d,
                                 wcat_a, wcat_b)
    return ((scores[0] + scores[1]) * 0.5).reshape(1)


# speedup vs baseline: 27.6614x; 1.1379x over previous
import functools

import jax
import jax.numpy as jnp
import numpy as np
from jax import lax
from jax.experimental import pallas as pl
from jax.experimental.pallas import tpu as pltpu
from jax.experimental.pallas import tpu_sc as plsc

N = 10000
E = 320000
D = 128
H = 8
DH = 16
NB = 2
NL = 2
NE = 8
NK = 16

HH = H // 2
HW = HH * DH

NC = 2
NSC = 16
NW = NC * NSC
EW = E // NW
C = 80
G = C // 16
NCH = EW // C
NPAD = 10240
RPT = NPAD // NSC
AW = HW + 16

NP = 10240
RW = NP // NW

_mesh = plsc.VectorSubcoreMesh(core_axis_name="c", subcore_axis_name="s")

_sc_params = pltpu.CompilerParams(needs_layout_passes=False,
                                  use_tc_tiling_on_sc=False)


@functools.partial(
    pl.kernel, mesh=_mesh,
    out_type=jax.ShapeDtypeStruct((NP, D), jnp.float32),
    scratch_types=[
        pltpu.VMEM((RW,), jnp.int32),
        pltpu.VMEM((RW, D), jnp.float32),
        pltpu.SemaphoreType.DMA,
    ],
    compiler_params=_sc_params,
)
def _gather_rows(table_hbm, idx_hbm, out_hbm, idxv, rows, sem):
    wid = lax.axis_index("c") * NSC + lax.axis_index("s")
    base = wid * RW
    pltpu.sync_copy(idx_hbm.at[pl.ds(base, RW)], idxv)
    pltpu.async_copy(table_hbm.at[idxv], rows, sem).wait()
    pltpu.sync_copy(rows, out_hbm.at[pl.ds(base, RW)])


@functools.partial(
    pl.kernel, mesh=_mesh,
    out_type=[
        jax.ShapeDtypeStruct((NC, NPAD, HW), jnp.float32),
        jax.ShapeDtypeStruct((NC, NPAD, 16), jnp.float32),
    ],
    scratch_types=[
        pltpu.VMEM((EW,), jnp.int32),
        pltpu.VMEM((EW,), jnp.int32),
        pltpu.VMEM((EW,), jnp.int32),
        pltpu.VMEM((N,), jnp.int32),
        pltpu.VMEM((NK, HW), jnp.float32),
        pltpu.VMEM((C,), jnp.int32),
        pltpu.VMEM((C,), jnp.int32),
        pltpu.VMEM((C,), jnp.int32),
        pltpu.VMEM((C,), jnp.int32),
        pltpu.VMEM((C, HW), jnp.float32),
        pltpu.VMEM((C, HW), jnp.float32),
        pltpu.VMEM((C, HW), jnp.float32),
        pltpu.VMEM((C, HW), jnp.float32),
        pltpu.VMEM((C, AW), jnp.float32),
        pltpu.VMEM((C, AW), jnp.float32),
        pltpu.VMEM_SHARED((NPAD, AW), jnp.float32),
        pltpu.SemaphoreType.DMA,
        pltpu.SemaphoreType.DMA,
        pltpu.SemaphoreType.DMA,
        pltpu.SemaphoreType.DMA,
    ],
    compiler_params=_sc_params,
)
def _edge_phase(kt_hbm, vt_hbm, qf_hbm, nk_hbm, src_hbm, dst_hbm, et_hbm,
                agg_out, den_out,
                srcb, dstb, etb, nkb, qtab,
                kvix0, kvix1, sctix0, sctix1,
                kbuf0, kbuf1, vbuf0, vbuf1, contrib0, contrib1,
                acc_sh, semg0, semg1, sems0, sems1):
    c = lax.axis_index("c")
    s = lax.axis_index("s")
    ebase = (c * NSC + s) * EW
    rbase = s * RPT

    zero16 = jnp.zeros((16,), jnp.float32)

    @pl.loop(0, C)
    def _(r):
        for cc in range(AW // 16):
            contrib0[r, pl.ds(cc * 16, 16)] = zero16
            contrib1[r, pl.ds(cc * 16, 16)] = zero16

    @pl.loop(0, RPT // C)
    def _(j):
        pltpu.sync_copy(contrib0, acc_sh.at[pl.ds(rbase + j * C, C)])
    plsc.subcore_barrier()

    pltpu.sync_copy(src_hbm.at[pl.ds(ebase, EW)], srcb)
    pltpu.sync_copy(dst_hbm.at[pl.ds(ebase, EW)], dstb)
    pltpu.sync_copy(et_hbm.at[pl.ds(ebase, EW)], etb)
    pltpu.sync_copy(nk_hbm, nkb)
    pltpu.sync_copy(qf_hbm, qtab)

    inv_sqrt_dh = float(1.0 / np.sqrt(DH))

    def prep(ch, kvixp, kbufp, vbufp, semp):
        off = ch * C
        for g in range(G):
            sl = pl.ds(off + g * 16, 16)
            kvixp[pl.ds(g * 16, 16)] = etb[sl] * N + srcb[sl]
        pltpu.async_copy(kt_hbm.at[kvixp], kbufp, semp)
        pltpu.async_copy(vt_hbm.at[kvixp], vbufp, semp)

    def consume(ch, kvixp, sctixp, kbufp, vbufp, contribp, semp, semsp):
        pltpu.make_async_copy(kt_hbm.at[kvixp], kbufp, semp).wait()
        pltpu.make_async_copy(vt_hbm.at[kvixp], vbufp, semp).wait()

        @pl.when(ch >= 2)
        def _():
            pltpu.make_async_copy(contribp, acc_sh.at[sctixp], semsp).wait()

        off = ch * C
        for g in range(G):
            sctixp[pl.ds(g * 16, 16)] = dstb[pl.ds(off + g * 16, 16)]

        @pl.loop(0, G)
        def _(g):
            rows = lax.iota(jnp.int32, 16) + g * 16
            dst16 = sctixp[pl.ds(g * 16, 16)]
            nk16 = plsc.load_gather(nkb, [dst16])
            for h in range(HH):
                acc = jnp.zeros((16,), jnp.float32)
                for p in range(DH):
                    col = jnp.full((16,), h * DH + p, jnp.int32)
                    qv = plsc.load_gather(qtab, [nk16, col])
                    kv = plsc.load_gather(kbufp, [rows, col])
                    acc = acc + qv * kv
                ex = jnp.exp(acc * inv_sqrt_dh)
                plsc.store_scatter(
                    contribp, [rows, jnp.full((16,), HW + h, jnp.int32)], ex)
                for p in range(DH):
                    col = jnp.full((16,), h * DH + p, jnp.int32)
                    vv = plsc.load_gather(vbufp, [rows, col])
                    plsc.store_scatter(contribp, [rows, col], vv * ex)

        pltpu.async_copy(contribp, acc_sh.at[sctixp], semsp, add=True)

    prep(0, kvix0, kbuf0, vbuf0, semg0)

    @pl.loop(0, (NCH - 1) // 2)
    def _(i):
        ch0 = i * 2
        prep(ch0 + 1, kvix1, kbuf1, vbuf1, semg1)
        consume(ch0, kvix0, sctix0, kbuf0, vbuf0, contrib0, semg0, sems0)
        prep(ch0 + 2, kvix0, kbuf0, vbuf0, semg0)
        consume(ch0 + 1, kvix1, sctix1, kbuf1, vbuf1, contrib1, semg1, sems1)

    consume(jnp.int32(NCH - 1), kvix0, sctix0, kbuf0, vbuf0, contrib0,
            semg0, sems0)
    pltpu.make_async_copy(contrib0, acc_sh.at[sctix0], sems0).wait()
    pltpu.make_async_copy(contrib1, acc_sh.at[sctix1], sems1).wait()

    plsc.subcore_barrier()

    @pl.loop(0, RPT // C)
    def _(j):
        r0 = rbase + j * C
        pltpu.sync_copy(acc_sh.at[pl.ds(r0, C), pl.ds(0, HW)],
                        agg_out.at[c, pl.ds(r0, C)])
        pltpu.sync_copy(acc_sh.at[pl.ds(r0, C), pl.ds(HW, 16)],
                        den_out.at[c, pl.ds(r0, C)])


TNP = 1000


def _proj_body(feat_ref, kw_ref, vw_ref, ka_ref, kb_ref, va_ref, vb_ref):
    f = feat_ref[...]
    dn = (((1,), (1,)), ((), ()))
    kt = lax.dot_general(f, kw_ref[0], dn, preferred_element_type=jnp.float32)
    vt = lax.dot_general(f, vw_ref[0], dn, preferred_element_type=jnp.float32)
    ka_ref[0] = kt[:, :HW]
    kb_ref[0] = kt[:, HW:]
    va_ref[0] = vt[:, :HW]
    vb_ref[0] = vt[:, HW:]


def _proj(feat, kwf, vwf):
    half = jax.ShapeDtypeStruct((NE, N, HW), jnp.float32)
    return pl.pallas_call(
        _proj_body,
        grid=(N // TNP, NE),
        in_specs=[
            pl.BlockSpec((TNP, D), lambda n, t: (n, 0)),
            pl.BlockSpec((1, D, D), lambda n, t: (t, 0, 0)),
            pl.BlockSpec((1, D, D), lambda n, t: (t, 0, 0)),
        ],
        out_specs=[
            pl.BlockSpec((1, TNP, HW), lambda n, t: (t, n, 0)),
            pl.BlockSpec((1, TNP, HW), lambda n, t: (t, n, 0)),
            pl.BlockSpec((1, TNP, HW), lambda n, t: (t, n, 0)),
            pl.BlockSpec((1, TNP, HW), lambda n, t: (t, n, 0)),
        ],
        out_shape=[half, half, half, half],
    )(feat, kwf, vwf)


TNO = 400


def _merge_normalize(agg_a_ref, agg_b_ref, den_a_ref, den_b_ref, nk_ref,
                     wcat_a_ref, wcat_b_ref):
    expand = (lax.broadcasted_iota(jnp.int32, (HH, HW), 0)
              == lax.broadcasted_iota(jnp.int32, (HH, HW), 1) // DH
              ).astype(jnp.float32)
    agg_a = agg_a_ref[0] + agg_a_ref[1]
    den4a = den_a_ref[0, :, :HH] + den_a_ref[1, :, :HH]
    aggna = agg_a / jnp.dot(den4a + 1e-9, expand,
                            preferred_element_type=jnp.float32)
    agg_b = agg_b_ref[0] + agg_b_ref[1]
    den4b = den_b_ref[0, :, :HH] + den_b_ref[1, :, :HH]
    aggnb = agg_b / jnp.dot(den4b + 1e-9, expand,
                            preferred_element_type=jnp.float32)
    big = (jnp.dot(aggna, wcat_a_ref[...], preferred_element_type=jnp.float32)
           + jnp.dot(aggnb, wcat_b_ref[...],
                     preferred_element_type=jnp.float32))
    nk = nk_ref[...]
    out = jnp.zeros((TNO, D), jnp.float32)
    for k in range(NK):
        m = (nk == k).astype(jnp.float32)
        out = out + m * big[:, k * D:(k + 1) * D]
    out = jnp.maximum(out, 0.0)
    mu = jnp.mean(out, axis=-1, keepdims=True)
    var = jnp.mean((out - mu) * (out - mu), axis=-1, keepdims=True)
    return (out - mu) * lax.rsqrt(var + 1e-5)


def _post_mid_body(agg_a_ref, agg_b_ref, den_a_ref, den_b_ref, nk_ref,
                   wcat_a_ref, wcat_b_ref, feat_ref):
    feat_ref[...] = _merge_normalize(agg_a_ref, agg_b_ref, den_a_ref,
                                     den_b_ref, nk_ref, wcat_a_ref, wcat_b_ref)


def _post_end_body(agg_a_ref, agg_b_ref, den_a_ref, den_b_ref, nk_ref,
                   wcat_a_ref, wcat_b_ref, prev_ref, twf_ref,
                   feat_ref, score_ref):
    f = _merge_normalize(agg_a_ref, agg_b_ref, den_a_ref, den_b_ref, nk_ref,
                         wcat_a_ref, wcat_b_ref) + prev_ref[...]
    feat_ref[...] = f
    nk = nk_ref[...]
    onehot = (nk == lax.broadcasted_iota(jnp.int32, (TNO, NK), 1)
              ).astype(jnp.float32)
    tw = jnp.dot(onehot, twf_ref[...], preferred_element_type=jnp.float32)

    @pl.when(pl.program_id(0) == 0)
    def _():
        score_ref[...] = jnp.zeros((1, 1), jnp.float32)

    score_ref[...] += (jnp.sum(f * tw) * (1.0 / N)).reshape(1, 1)


_POST_SPECS = [
    pl.BlockSpec((NC, TNO, HW), lambda n: (0, n, 0)),
    pl.BlockSpec((NC, TNO, HW), lambda n: (0, n, 0)),
    pl.BlockSpec((NC, TNO, 16), lambda n: (0, n, 0)),
    pl.BlockSpec((NC, TNO, 16), lambda n: (0, n, 0)),
    pl.BlockSpec((TNO, 1), lambda n: (n, 0)),
    pl.BlockSpec((HW, NK * D), lambda n: (0, 0)),
    pl.BlockSpec((HW, NK * D), lambda n: (0, 0)),
]


def _post_mid(agg_a, agg_b, den_a, den_b, nk2d, wcat_a, wcat_b):
    return pl.pallas_call(
        _post_mid_body,
        grid=(N // TNO,),
        in_specs=_POST_SPECS,
        out_specs=pl.BlockSpec((TNO, D), lambda n: (n, 0)),
        out_shape=jax.ShapeDtypeStruct((N, D), jnp.float32),
    )(agg_a, agg_b, den_a, den_b, nk2d, wcat_a, wcat_b)


def _post_end(agg_a, agg_b, den_a, den_b, nk2d, wcat_a, wcat_b, prev, twf):
    return pl.pallas_call(
        _post_end_body,
        grid=(N // TNO,),
        in_specs=_POST_SPECS + [
            pl.BlockSpec((TNO, D), lambda n: (n, 0)),
            pl.BlockSpec((NK, D), lambda n: (0, 0)),
        ],
        out_specs=[
            pl.BlockSpec((TNO, D), lambda n: (n, 0)),
            pl.BlockSpec((1, 1), lambda n: (0, 0)),
        ],
        out_shape=[
            jax.ShapeDtypeStruct((N, D), jnp.float32),
            jax.ShapeDtypeStruct((1, 1), jnp.float32),
        ],
    )(agg_a, agg_b, den_a, den_b, nk2d, wcat_a, wcat_b, prev, twf)


def kernel(emb, key_w, val_w, query, node_w, target_w,
           node_strings, node_keys, edge_index, edge_types):
    src = edge_index[0].astype(jnp.int32)
    dst = edge_index[1].astype(jnp.int32)
    et = edge_types.astype(jnp.int32)
    nstr = node_strings.astype(jnp.int32)
    nkey = node_keys.astype(jnp.int32)

    pad = jnp.zeros((NP - N,), jnp.int32)
    nsp = jnp.concatenate([nstr, pad])
    nk2d = nkey[:, None]

    feat = _gather_rows(emb, nsp)[:N]
    prev = feat
    scores = []
    for i in range(NB):
        qf = query[i].reshape(NK, D)
        q_a = qf[:, :HW]
        q_b = qf[:, HW:]
        wcat = node_w[i].reshape(NK, D, D).transpose(1, 0, 2).reshape(D, NK * D)
        wcat_a = wcat[:HW]
        wcat_b = wcat[HW:]
        twf = target_w[i].reshape(NK, D)
        kwf = key_w[i].reshape(NE, D, D)
        vwf = val_w[i].reshape(NE, D, D)
        for l in range(NL):
            ka, kb, va, vb = _proj(feat, kwf, vwf)
            agg_a, den_a = _edge_phase(ka.reshape(NE * N, HW),
                                       va.reshape(NE * N, HW),
                                       q_a, nkey, src, dst, et)
            agg_b, den_b = _edge_phase(kb.reshape(NE * N, HW),
                                       vb.reshape(NE * N, HW),
                                       q_b, nkey, src, dst, et)
            if l == NL - 1:
                feat, sc = _post_end(agg_a, agg_b, den_a, den_b, nk2d,
                                     wcat_a, wcat_b, prev, twf)
                prev = feat
                scores.append(sc[0, 0])
            else:
                feat = _post_mid(agg_a, agg_b, den_a, den_b, nk2d,
                                 wcat_a, wcat_b)
    return ((scores[0] + scores[1]) * 0.5).reshape(1)


# speedup vs baseline: 28.5493x; 1.0321x over previous
import functools

import jax
import jax.numpy as jnp
import numpy as np
from jax import lax
from jax.experimental import pallas as pl
from jax.experimental.pallas import tpu as pltpu
from jax.experimental.pallas import tpu_sc as plsc

N = 10000
E = 320000
D = 128
H = 8
DH = 16
NB = 2
NL = 2
NE = 8
NK = 16

HH = H // 2
HW = HH * DH

NC = 2
NSC = 16
NW = NC * NSC
EW = E // NW
C = 80
G = C // 16
NCH = EW // C
NPAD = 10240
RPT = NPAD // NSC
AW = HW + 16

NP = 10240
RW = NP // NW

_mesh = plsc.VectorSubcoreMesh(core_axis_name="c", subcore_axis_name="s")

_sc_params = pltpu.CompilerParams(needs_layout_passes=False,
                                  use_tc_tiling_on_sc=False)


@functools.partial(
    pl.kernel, mesh=_mesh,
    out_type=jax.ShapeDtypeStruct((NP, D), jnp.float32),
    scratch_types=[
        pltpu.VMEM((RW,), jnp.int32),
        pltpu.VMEM((RW, D), jnp.float32),
        pltpu.SemaphoreType.DMA,
    ],
    compiler_params=_sc_params,
)
def _gather_rows(table_hbm, idx_hbm, out_hbm, idxv, rows, sem):
    wid = lax.axis_index("c") * NSC + lax.axis_index("s")
    base = wid * RW
    pltpu.sync_copy(idx_hbm.at[pl.ds(base, RW)], idxv)
    pltpu.async_copy(table_hbm.at[idxv], rows, sem).wait()
    pltpu.sync_copy(rows, out_hbm.at[pl.ds(base, RW)])


@functools.partial(
    pl.kernel, mesh=_mesh,
    out_type=[
        jax.ShapeDtypeStruct((NC, NPAD, HW), jnp.float32),
        jax.ShapeDtypeStruct((NC, NPAD, 16), jnp.float32),
    ],
    scratch_types=[
        pltpu.VMEM((EW,), jnp.int32),
        pltpu.VMEM((EW,), jnp.int32),
        pltpu.VMEM((EW,), jnp.int32),
        pltpu.VMEM((N,), jnp.int32),
        pltpu.VMEM((NK, HW), jnp.float32),
        pltpu.VMEM((C,), jnp.int32),
        pltpu.VMEM((C,), jnp.int32),
        pltpu.VMEM((C,), jnp.int32),
        pltpu.VMEM((C,), jnp.int32),
        pltpu.VMEM((C, HW), jnp.float32),
        pltpu.VMEM((C, HW), jnp.float32),
        pltpu.VMEM((C, HW), jnp.float32),
        pltpu.VMEM((C, HW), jnp.float32),
        pltpu.VMEM((C, AW), jnp.float32),
        pltpu.VMEM((C, AW), jnp.float32),
        pltpu.VMEM_SHARED((NPAD, AW), jnp.float32),
        pltpu.SemaphoreType.DMA,
        pltpu.SemaphoreType.DMA,
        pltpu.SemaphoreType.DMA,
        pltpu.SemaphoreType.DMA,
    ],
    compiler_params=_sc_params,
)
def _edge_phase(kt_hbm, vt_hbm, qf_hbm, nk_hbm, src_hbm, dst_hbm, et_hbm,
                agg_out, den_out,
                srcb, dstb, etb, nkb, qtab,
                kvix0, kvix1, sctix0, sctix1,
                kbuf0, kbuf1, vbuf0, vbuf1, contrib0, contrib1,
                acc_sh, semg0, semg1, sems0, sems1):
    c = lax.axis_index("c")
    s = lax.axis_index("s")
    ebase = (c * NSC + s) * EW
    rbase = s * RPT

    zero16 = jnp.zeros((16,), jnp.float32)

    @pl.loop(0, C)
    def _(r):
        for cc in range(AW // 16):
            contrib0[r, pl.ds(cc * 16, 16)] = zero16
            contrib1[r, pl.ds(cc * 16, 16)] = zero16

    @pl.loop(0, RPT // C)
    def _(j):
        pltpu.sync_copy(contrib0, acc_sh.at[pl.ds(rbase + j * C, C)])
    plsc.subcore_barrier()

    pltpu.sync_copy(src_hbm.at[pl.ds(ebase, EW)], srcb)
    pltpu.sync_copy(dst_hbm.at[pl.ds(ebase, EW)], dstb)
    pltpu.sync_copy(et_hbm.at[pl.ds(ebase, EW)], etb)
    pltpu.sync_copy(nk_hbm, nkb)
    pltpu.sync_copy(qf_hbm, qtab)

    inv_sqrt_dh = float(1.0 / np.sqrt(DH))

    @pl.loop(0, NK)
    def _(k):
        for cc in range(HW // 16):
            qtab[k, pl.ds(cc * 16, 16)] = qtab[k, pl.ds(cc * 16, 16)] * inv_sqrt_dh

    def prep(ch, kvixp, kbufp, vbufp, semp):
        off = ch * C
        for g in range(G):
            sl = pl.ds(off + g * 16, 16)
            kvixp[pl.ds(g * 16, 16)] = etb[sl] * N + srcb[sl]
        pltpu.async_copy(kt_hbm.at[kvixp], kbufp, semp)
        pltpu.async_copy(vt_hbm.at[kvixp], vbufp, semp)

    def consume(ch, kvixp, sctixp, kbufp, vbufp, contribp, semp, semsp):
        pltpu.make_async_copy(kt_hbm.at[kvixp], kbufp, semp).wait()
        pltpu.make_async_copy(vt_hbm.at[kvixp], vbufp, semp).wait()

        @pl.when(ch >= 2)
        def _():
            pltpu.make_async_copy(contribp, acc_sh.at[sctixp], semsp).wait()

        off = ch * C
        for g in range(G):
            sctixp[pl.ds(g * 16, 16)] = dstb[pl.ds(off + g * 16, 16)]

        @pl.loop(0, G)
        def _(g):
            rows = lax.iota(jnp.int32, 16) + g * 16
            dst16 = sctixp[pl.ds(g * 16, 16)]
            nk16 = plsc.load_gather(nkb, [dst16])
            for h in range(HH):
                parts = [None, None, None, None]
                for p in range(DH):
                    col = jnp.full((16,), h * DH + p, jnp.int32)
                    qv = plsc.load_gather(qtab, [nk16, col])
                    kv = plsc.load_gather(kbufp, [rows, col])
                    prod = qv * kv
                    lane = p % 4
                    parts[lane] = prod if parts[lane] is None else parts[lane] + prod
                ex = jnp.exp((parts[0] + parts[1]) + (parts[2] + parts[3]))
                plsc.store_scatter(
                    contribp, [rows, jnp.full((16,), HW + h, jnp.int32)], ex)
                for p in range(DH):
                    col = jnp.full((16,), h * DH + p, jnp.int32)
                    vv = plsc.load_gather(vbufp, [rows, col])
                    plsc.store_scatter(contribp, [rows, col], vv * ex)

        pltpu.async_copy(contribp, acc_sh.at[sctixp], semsp, add=True)

    prep(0, kvix0, kbuf0, vbuf0, semg0)

    @pl.loop(0, (NCH - 1) // 2)
    def _(i):
        ch0 = i * 2
        prep(ch0 + 1, kvix1, kbuf1, vbuf1, semg1)
        consume(ch0, kvix0, sctix0, kbuf0, vbuf0, contrib0, semg0, sems0)
        prep(ch0 + 2, kvix0, kbuf0, vbuf0, semg0)
        consume(ch0 + 1, kvix1, sctix1, kbuf1, vbuf1, contrib1, semg1, sems1)

    consume(jnp.int32(NCH - 1), kvix0, sctix0, kbuf0, vbuf0, contrib0,
            semg0, sems0)
    pltpu.make_async_copy(contrib0, acc_sh.at[sctix0], sems0).wait()
    pltpu.make_async_copy(contrib1, acc_sh.at[sctix1], sems1).wait()

    plsc.subcore_barrier()

    @pl.loop(0, RPT // C)
    def _(j):
        r0 = rbase + j * C
        pltpu.sync_copy(acc_sh.at[pl.ds(r0, C), pl.ds(0, HW)],
                        agg_out.at[c, pl.ds(r0, C)])
        pltpu.sync_copy(acc_sh.at[pl.ds(r0, C), pl.ds(HW, 16)],
                        den_out.at[c, pl.ds(r0, C)])


TNP = 1000


def _proj_body(feat_ref, kw_ref, vw_ref, ka_ref, kb_ref, va_ref, vb_ref):
    f = feat_ref[...]
    dn = (((1,), (1,)), ((), ()))
    kt = lax.dot_general(f, kw_ref[0], dn, preferred_element_type=jnp.float32)
    vt = lax.dot_general(f, vw_ref[0], dn, preferred_element_type=jnp.float32)
    ka_ref[0] = kt[:, :HW]
    kb_ref[0] = kt[:, HW:]
    va_ref[0] = vt[:, :HW]
    vb_ref[0] = vt[:, HW:]


def _proj(feat, kwf, vwf):
    half = jax.ShapeDtypeStruct((NE, N, HW), jnp.float32)
    return pl.pallas_call(
        _proj_body,
        grid=(N // TNP, NE),
        in_specs=[
            pl.BlockSpec((TNP, D), lambda n, t: (n, 0)),
            pl.BlockSpec((1, D, D), lambda n, t: (t, 0, 0)),
            pl.BlockSpec((1, D, D), lambda n, t: (t, 0, 0)),
        ],
        out_specs=[
            pl.BlockSpec((1, TNP, HW), lambda n, t: (t, n, 0)),
            pl.BlockSpec((1, TNP, HW), lambda n, t: (t, n, 0)),
            pl.BlockSpec((1, TNP, HW), lambda n, t: (t, n, 0)),
            pl.BlockSpec((1, TNP, HW), lambda n, t: (t, n, 0)),
        ],
        out_shape=[half, half, half, half],
    )(feat, kwf, vwf)


TNO = 400


def _merge_normalize(agg_a_ref, agg_b_ref, den_a_ref, den_b_ref, nk_ref,
                     wcat_a_ref, wcat_b_ref):
    expand = (lax.broadcasted_iota(jnp.int32, (HH, HW), 0)
              == lax.broadcasted_iota(jnp.int32, (HH, HW), 1) // DH
              ).astype(jnp.float32)
    agg_a = agg_a_ref[0] + agg_a_ref[1]
    den4a = den_a_ref[0, :, :HH] + den_a_ref[1, :, :HH]
    aggna = agg_a / jnp.dot(den4a + 1e-9, expand,
                            preferred_element_type=jnp.float32)
    agg_b = agg_b_ref[0] + agg_b_ref[1]
    den4b = den_b_ref[0, :, :HH] + den_b_ref[1, :, :HH]
    aggnb = agg_b / jnp.dot(den4b + 1e-9, expand,
                            preferred_element_type=jnp.float32)
    big = (jnp.dot(aggna, wcat_a_ref[...], preferred_element_type=jnp.float32)
           + jnp.dot(aggnb, wcat_b_ref[...],
                     preferred_element_type=jnp.float32))
    nk = nk_ref[...]
    out = jnp.zeros((TNO, D), jnp.float32)
    for k in range(NK):
        m = (nk == k).astype(jnp.float32)
        out = out + m * big[:, k * D:(k + 1) * D]
    out = jnp.maximum(out, 0.0)
    mu = jnp.mean(out, axis=-1, keepdims=True)
    var = jnp.mean((out - mu) * (out - mu), axis=-1, keepdims=True)
    return (out - mu) * lax.rsqrt(var + 1e-5)


def _post_mid_body(agg_a_ref, agg_b_ref, den_a_ref, den_b_ref, nk_ref,
                   wcat_a_ref, wcat_b_ref, feat_ref):
    feat_ref[...] = _merge_normalize(agg_a_ref, agg_b_ref, den_a_ref,
                                     den_b_ref, nk_ref, wcat_a_ref, wcat_b_ref)


def _post_end_body(agg_a_ref, agg_b_ref, den_a_ref, den_b_ref, nk_ref,
                   wcat_a_ref, wcat_b_ref, prev_ref, twf_ref,
                   feat_ref, score_ref):
    f = _merge_normalize(agg_a_ref, agg_b_ref, den_a_ref, den_b_ref, nk_ref,
                         wcat_a_ref, wcat_b_ref) + prev_ref[...]
    feat_ref[...] = f
    nk = nk_ref[...]
    onehot = (nk == lax.broadcasted_iota(jnp.int32, (TNO, NK), 1)
              ).astype(jnp.float32)
    tw = jnp.dot(onehot, twf_ref[...], preferred_element_type=jnp.float32)

    @pl.when(pl.program_id(0) == 0)
    def _():
        score_ref[...] = jnp.zeros((1, 1), jnp.float32)

    score_ref[...] += (jnp.sum(f * tw) * (1.0 / N)).reshape(1, 1)


_POST_SPECS = [
    pl.BlockSpec((NC, TNO, HW), lambda n: (0, n, 0)),
    pl.BlockSpec((NC, TNO, HW), lambda n: (0, n, 0)),
    pl.BlockSpec((NC, TNO, 16), lambda n: (0, n, 0)),
    pl.BlockSpec((NC, TNO, 16), lambda n: (0, n, 0)),
    pl.BlockSpec((TNO, 1), lambda n: (n, 0)),
    pl.BlockSpec((HW, NK * D), lambda n: (0, 0)),
    pl.BlockSpec((HW, NK * D), lambda n: (0, 0)),
]


def _post_mid(agg_a, agg_b, den_a, den_b, nk2d, wcat_a, wcat_b):
    return pl.pallas_call(
        _post_mid_body,
        grid=(N // TNO,),
        in_specs=_POST_SPECS,
        out_specs=pl.BlockSpec((TNO, D), lambda n: (n, 0)),
        out_shape=jax.ShapeDtypeStruct((N, D), jnp.float32),
    )(agg_a, agg_b, den_a, den_b, nk2d, wcat_a, wcat_b)


def _post_end(agg_a, agg_b, den_a, den_b, nk2d, wcat_a, wcat_b, prev, twf):
    return pl.pallas_call(
        _post_end_body,
        grid=(N // TNO,),
        in_specs=_POST_SPECS + [
            pl.BlockSpec((TNO, D), lambda n: (n, 0)),
            pl.BlockSpec((NK, D), lambda n: (0, 0)),
        ],
        out_specs=[
            pl.BlockSpec((TNO, D), lambda n: (n, 0)),
            pl.BlockSpec((1, 1), lambda n: (0, 0)),
        ],
        out_shape=[
            jax.ShapeDtypeStruct((N, D), jnp.float32),
            jax.ShapeDtypeStruct((1, 1), jnp.float32),
        ],
    )(agg_a, agg_b, den_a, den_b, nk2d, wcat_a, wcat_b, prev, twf)


def kernel(emb, key_w, val_w, query, node_w, target_w,
           node_strings, node_keys, edge_index, edge_types):
    src = edge_index[0].astype(jnp.int32)
    dst = edge_index[1].astype(jnp.int32)
    et = edge_types.astype(jnp.int32)
    nstr = node_strings.astype(jnp.int32)
    nkey = node_keys.astype(jnp.int32)

    pad = jnp.zeros((NP - N,), jnp.int32)
    nsp = jnp.concatenate([nstr, pad])
    nk2d = nkey[:, None]

    feat = _gather_rows(emb, nsp)[:N]
    prev = feat
    scores = []
    for i in range(NB):
        qf = query[i].reshape(NK, D)
        q_a = qf[:, :HW]
        q_b = qf[:, HW:]
        wcat = node_w[i].reshape(NK, D, D).transpose(1, 0, 2).reshape(D, NK * D)
        wcat_a = wcat[:HW]
        wcat_b = wcat[HW:]
        twf = target_w[i].reshape(NK, D)
        kwf = key_w[i].reshape(NE, D, D)
        vwf = val_w[i].reshape(NE, D, D)
        for l in range(NL):
            ka, kb, va, vb = _proj(feat, kwf, vwf)
            agg_a, den_a = _edge_phase(ka.reshape(NE * N, HW),
                                       va.reshape(NE * N, HW),
                                       q_a, nkey, src, dst, et)
            agg_b, den_b = _edge_phase(kb.reshape(NE * N, HW),
                                       vb.reshape(NE * N, HW),
                                       q_b, nkey, src, dst, et)
            if l == NL - 1:
                feat, sc = _post_end(agg_a, agg_b, den_a, den_b, nk2d,
                                     wcat_a, wcat_b, prev, twf)
                prev = feat
                scores.append(sc[0, 0])
            else:
                feat = _post_mid(agg_a, agg_b, den_a, den_b, nk2d,
                                 wcat_a, wcat_b)
    return ((scores[0] + scores[1]) * 0.5).reshape(1)


# speedup vs baseline: 28.5506x; 1.0000x over previous
import functools

import jax
import jax.numpy as jnp
import numpy as np
from jax import lax
from jax.experimental import pallas as pl
from jax.experimental.pallas import tpu as pltpu
from jax.experimental.pallas import tpu_sc as plsc

N = 10000
E = 320000
D = 128
H = 8
DH = 16
NB = 2
NL = 2
NE = 8
NK = 16

HH = H // 2
HW = HH * DH

NC = 2
NSC = 16
NW = NC * NSC
EW = E // NW
C = 80
G = C // 16
NCH = EW // C
NPAD = 10240
RPT = NPAD // NSC
AW = HW + 16

NP = 10240
RW = NP // NW

_mesh = plsc.VectorSubcoreMesh(core_axis_name="c", subcore_axis_name="s")

_sc_params = pltpu.CompilerParams(needs_layout_passes=False,
                                  use_tc_tiling_on_sc=False)


@functools.partial(
    pl.kernel, mesh=_mesh,
    out_type=jax.ShapeDtypeStruct((NP, D), jnp.float32),
    scratch_types=[
        pltpu.VMEM((RW,), jnp.int32),
        pltpu.VMEM((RW, D), jnp.float32),
        pltpu.SemaphoreType.DMA,
    ],
    compiler_params=_sc_params,
)
def _gather_rows(table_hbm, idx_hbm, out_hbm, idxv, rows, sem):
    wid = lax.axis_index("c") * NSC + lax.axis_index("s")
    base = wid * RW
    pltpu.sync_copy(idx_hbm.at[pl.ds(base, RW)], idxv)
    pltpu.async_copy(table_hbm.at[idxv], rows, sem).wait()
    pltpu.sync_copy(rows, out_hbm.at[pl.ds(base, RW)])


@functools.partial(
    pl.kernel, mesh=_mesh,
    out_type=[
        jax.ShapeDtypeStruct((NC, NPAD, HW), jnp.float32),
        jax.ShapeDtypeStruct((NC, NPAD, 16), jnp.float32),
    ],
    scratch_types=[
        pltpu.VMEM((EW,), jnp.int32),
        pltpu.VMEM((EW,), jnp.int32),
        pltpu.VMEM((EW,), jnp.int32),
        pltpu.VMEM((N,), jnp.int32),
        pltpu.VMEM((NK, HW), jnp.float32),
        pltpu.VMEM((C,), jnp.int32),
        pltpu.VMEM((C,), jnp.int32),
        pltpu.VMEM((C,), jnp.int32),
        pltpu.VMEM((C,), jnp.int32),
        pltpu.VMEM((C, HW), jnp.float32),
        pltpu.VMEM((C, HW), jnp.float32),
        pltpu.VMEM((C, HW), jnp.float32),
        pltpu.VMEM((C, HW), jnp.float32),
        pltpu.VMEM((C, AW), jnp.float32),
        pltpu.VMEM((C, AW), jnp.float32),
        pltpu.VMEM_SHARED((NPAD, AW), jnp.float32),
        pltpu.SemaphoreType.DMA,
        pltpu.SemaphoreType.DMA,
        pltpu.SemaphoreType.DMA,
        pltpu.SemaphoreType.DMA,
    ],
    compiler_params=_sc_params,
)
def _edge_phase(kt_hbm, vt_hbm, qf_hbm, nk_hbm, src_hbm, dst_hbm, et_hbm,
                agg_out, den_out,
                srcb, dstb, etb, nkb, qtab,
                kvix0, kvix1, sctix0, sctix1,
                kbuf0, kbuf1, vbuf0, vbuf1, contrib0, contrib1,
                acc_sh, semg0, semg1, sems0, sems1):
    c = lax.axis_index("c")
    s = lax.axis_index("s")
    ebase = (c * NSC + s) * EW
    rbase = s * RPT

    zero16 = jnp.zeros((16,), jnp.float32)

    @pl.loop(0, C)
    def _(r):
        for cc in range(AW // 16):
            contrib0[r, pl.ds(cc * 16, 16)] = zero16
            contrib1[r, pl.ds(cc * 16, 16)] = zero16

    @pl.loop(0, RPT // C)
    def _(j):
        pltpu.sync_copy(contrib0, acc_sh.at[pl.ds(rbase + j * C, C)])
    plsc.subcore_barrier()

    pltpu.sync_copy(src_hbm.at[pl.ds(ebase, EW)], srcb)
    pltpu.sync_copy(dst_hbm.at[pl.ds(ebase, EW)], dstb)
    pltpu.sync_copy(et_hbm.at[pl.ds(ebase, EW)], etb)
    pltpu.sync_copy(nk_hbm, nkb)
    pltpu.sync_copy(qf_hbm, qtab)

    inv_sqrt_dh = float(1.0 / np.sqrt(DH))

    @pl.loop(0, NK)
    def _(k):
        for cc in range(HW // 16):
            qtab[k, pl.ds(cc * 16, 16)] = qtab[k, pl.ds(cc * 16, 16)] * inv_sqrt_dh

    def prep(ch, kvixp, kbufp, vbufp, semp):
        off = ch * C
        for g in range(G):
            sl = pl.ds(off + g * 16, 16)
            kvixp[pl.ds(g * 16, 16)] = etb[sl] * N + srcb[sl]
        pltpu.async_copy(kt_hbm.at[kvixp], kbufp, semp)
        pltpu.async_copy(vt_hbm.at[kvixp], vbufp, semp)

    def consume(ch, kvixp, sctixp, kbufp, vbufp, contribp, semp, semsp):
        pltpu.make_async_copy(kt_hbm.at[kvixp], kbufp, semp).wait()
        pltpu.make_async_copy(vt_hbm.at[kvixp], vbufp, semp).wait()

        @pl.when(ch >= 2)
        def _():
            pltpu.make_async_copy(contribp, acc_sh.at[sctixp], semsp).wait()

        off = ch * C
        for g in range(G):
            sctixp[pl.ds(g * 16, 16)] = dstb[pl.ds(off + g * 16, 16)]

        @plsc.parallel_loop(0, G)
        def _(g):
            rows = lax.iota(jnp.int32, 16) + g * 16
            dst16 = sctixp[pl.ds(g * 16, 16)]
            nk16 = plsc.load_gather(nkb, [dst16])
            for h in range(HH):
                parts = [None, None, None, None]
                for p in range(DH):
                    col = jnp.full((16,), h * DH + p, jnp.int32)
                    qv = plsc.load_gather(qtab, [nk16, col])
                    kv = plsc.load_gather(kbufp, [rows, col])
                    prod = qv * kv
                    lane = p % 4
                    parts[lane] = prod if parts[lane] is None else parts[lane] + prod
                ex = jnp.exp((parts[0] + parts[1]) + (parts[2] + parts[3]))
                plsc.store_scatter(
                    contribp, [rows, jnp.full((16,), HW + h, jnp.int32)], ex)
                for p in range(DH):
                    col = jnp.full((16,), h * DH + p, jnp.int32)
                    vv = plsc.load_gather(vbufp, [rows, col])
                    plsc.store_scatter(contribp, [rows, col], vv * ex)

        pltpu.async_copy(contribp, acc_sh.at[sctixp], semsp, add=True)

    prep(0, kvix0, kbuf0, vbuf0, semg0)

    @pl.loop(0, (NCH - 1) // 2)
    def _(i):
        ch0 = i * 2
        prep(ch0 + 1, kvix1, kbuf1, vbuf1, semg1)
        consume(ch0, kvix0, sctix0, kbuf0, vbuf0, contrib0, semg0, sems0)
        prep(ch0 + 2, kvix0, kbuf0, vbuf0, semg0)
        consume(ch0 + 1, kvix1, sctix1, kbuf1, vbuf1, contrib1, semg1, sems1)

    consume(jnp.int32(NCH - 1), kvix0, sctix0, kbuf0, vbuf0, contrib0,
            semg0, sems0)
    pltpu.make_async_copy(contrib0, acc_sh.at[sctix0], sems0).wait()
    pltpu.make_async_copy(contrib1, acc_sh.at[sctix1], sems1).wait()

    plsc.subcore_barrier()

    @pl.loop(0, RPT // C)
    def _(j):
        r0 = rbase + j * C
        pltpu.sync_copy(acc_sh.at[pl.ds(r0, C), pl.ds(0, HW)],
                        agg_out.at[c, pl.ds(r0, C)])
        pltpu.sync_copy(acc_sh.at[pl.ds(r0, C), pl.ds(HW, 16)],
                        den_out.at[c, pl.ds(r0, C)])


TNP = 1000


def _proj_body(feat_ref, kw_ref, vw_ref, ka_ref, kb_ref, va_ref, vb_ref):
    f = feat_ref[...]
    dn = (((1,), (1,)), ((), ()))
    kt = lax.dot_general(f, kw_ref[0], dn, preferred_element_type=jnp.float32)
    vt = lax.dot_general(f, vw_ref[0], dn, preferred_element_type=jnp.float32)
    ka_ref[0] = kt[:, :HW]
    kb_ref[0] = kt[:, HW:]
    va_ref[0] = vt[:, :HW]
    vb_ref[0] = vt[:, HW:]


def _proj(feat, kwf, vwf):
    half = jax.ShapeDtypeStruct((NE, N, HW), jnp.float32)
    return pl.pallas_call(
        _proj_body,
        grid=(N // TNP, NE),
        in_specs=[
            pl.BlockSpec((TNP, D), lambda n, t: (n, 0)),
            pl.BlockSpec((1, D, D), lambda n, t: (t, 0, 0)),
            pl.BlockSpec((1, D, D), lambda n, t: (t, 0, 0)),
        ],
        out_specs=[
            pl.BlockSpec((1, TNP, HW), lambda n, t: (t, n, 0)),
            pl.BlockSpec((1, TNP, HW), lambda n, t: (t, n, 0)),
            pl.BlockSpec((1, TNP, HW), lambda n, t: (t, n, 0)),
            pl.BlockSpec((1, TNP, HW), lambda n, t: (t, n, 0)),
        ],
        out_shape=[half, half, half, half],
    )(feat, kwf, vwf)


TNO = 400


def _merge_normalize(agg_a_ref, agg_b_ref, den_a_ref, den_b_ref, nk_ref,
                     wcat_a_ref, wcat_b_ref):
    expand = (lax.broadcasted_iota(jnp.int32, (HH, HW), 0)
              == lax.broadcasted_iota(jnp.int32, (HH, HW), 1) // DH
              ).astype(jnp.float32)
    agg_a = agg_a_ref[0] + agg_a_ref[1]
    den4a = den_a_ref[0, :, :HH] + den_a_ref[1, :, :HH]
    aggna = agg_a / jnp.dot(den4a + 1e-9, expand,
                            preferred_element_type=jnp.float32)
    agg_b = agg_b_ref[0] + agg_b_ref[1]
    den4b = den_b_ref[0, :, :HH] + den_b_ref[1, :, :HH]
    aggnb = agg_b / jnp.dot(den4b + 1e-9, expand,
                            preferred_element_type=jnp.float32)
    big = (jnp.dot(aggna, wcat_a_ref[...], preferred_element_type=jnp.float32)
           + jnp.dot(aggnb, wcat_b_ref[...],
                     preferred_element_type=jnp.float32))
    nk = nk_ref[...]
    out = jnp.zeros((TNO, D), jnp.float32)
    for k in range(NK):
        m = (nk == k).astype(jnp.float32)
        out = out + m * big[:, k * D:(k + 1) * D]
    out = jnp.maximum(out, 0.0)
    mu = jnp.mean(out, axis=-1, keepdims=True)
    var = jnp.mean((out - mu) * (out - mu), axis=-1, keepdims=True)
    return (out - mu) * lax.rsqrt(var + 1e-5)


def _post_mid_body(agg_a_ref, agg_b_ref, den_a_ref, den_b_ref, nk_ref,
                   wcat_a_ref, wcat_b_ref, feat_ref):
    feat_ref[...] = _merge_normalize(agg_a_ref, agg_b_ref, den_a_ref,
                                     den_b_ref, nk_ref, wcat_a_ref, wcat_b_ref)


def _post_end_body(agg_a_ref, agg_b_ref, den_a_ref, den_b_ref, nk_ref,
                   wcat_a_ref, wcat_b_ref, prev_ref, twf_ref,
                   feat_ref, score_ref):
    f = _merge_normalize(agg_a_ref, agg_b_ref, den_a_ref, den_b_ref, nk_ref,
                         wcat_a_ref, wcat_b_ref) + prev_ref[...]
    feat_ref[...] = f
    nk = nk_ref[...]
    onehot = (nk == lax.broadcasted_iota(jnp.int32, (TNO, NK), 1)
              ).astype(jnp.float32)
    tw = jnp.dot(onehot, twf_ref[...], preferred_element_type=jnp.float32)

    @pl.when(pl.program_id(0) == 0)
    def _():
        score_ref[...] = jnp.zeros((1, 1), jnp.float32)

    score_ref[...] += (jnp.sum(f * tw) * (1.0 / N)).reshape(1, 1)


_POST_SPECS = [
    pl.BlockSpec((NC, TNO, HW), lambda n: (0, n, 0)),
    pl.BlockSpec((NC, TNO, HW), lambda n: (0, n, 0)),
    pl.BlockSpec((NC, TNO, 16), lambda n: (0, n, 0)),
    pl.BlockSpec((NC, TNO, 16), lambda n: (0, n, 0)),
    pl.BlockSpec((TNO, 1), lambda n: (n, 0)),
    pl.BlockSpec((HW, NK * D), lambda n: (0, 0)),
    pl.BlockSpec((HW, NK * D), lambda n: (0, 0)),
]


def _post_mid(agg_a, agg_b, den_a, den_b, nk2d, wcat_a, wcat_b):
    return pl.pallas_call(
        _post_mid_body,
        grid=(N // TNO,),
        in_specs=_POST_SPECS,
        out_specs=pl.BlockSpec((TNO, D), lambda n: (n, 0)),
        out_shape=jax.ShapeDtypeStruct((N, D), jnp.float32),
    )(agg_a, agg_b, den_a, den_b, nk2d, wcat_a, wcat_b)


def _post_end(agg_a, agg_b, den_a, den_b, nk2d, wcat_a, wcat_b, prev, twf):
    return pl.pallas_call(
        _post_end_body,
        grid=(N // TNO,),
        in_specs=_POST_SPECS + [
            pl.BlockSpec((TNO, D), lambda n: (n, 0)),
            pl.BlockSpec((NK, D), lambda n: (0, 0)),
        ],
        out_specs=[
            pl.BlockSpec((TNO, D), lambda n: (n, 0)),
            pl.BlockSpec((1, 1), lambda n: (0, 0)),
        ],
        out_shape=[
            jax.ShapeDtypeStruct((N, D), jnp.float32),
            jax.ShapeDtypeStruct((1, 1), jnp.float32),
        ],
    )(agg_a, agg_b, den_a, den_b, nk2d, wcat_a, wcat_b, prev, twf)


def kernel(emb, key_w, val_w, query, node_w, target_w,
           node_strings, node_keys, edge_index, edge_types):
    src = edge_index[0].astype(jnp.int32)
    dst = edge_index[1].astype(jnp.int32)
    et = edge_types.astype(jnp.int32)
    nstr = node_strings.astype(jnp.int32)
    nkey = node_keys.astype(jnp.int32)

    pad = jnp.zeros((NP - N,), jnp.int32)
    nsp = jnp.concatenate([nstr, pad])
    nk2d = nkey[:, None]

    feat = _gather_rows(emb, nsp)[:N]
    prev = feat
    scores = []
    for i in range(NB):
        qf = query[i].reshape(NK, D)
        q_a = qf[:, :HW]
        q_b = qf[:, HW:]
        wcat = node_w[i].reshape(NK, D, D).transpose(1, 0, 2).reshape(D, NK * D)
        wcat_a = wcat[:HW]
        wcat_b = wcat[HW:]
        twf = target_w[i].reshape(NK, D)
        kwf = key_w[i].reshape(NE, D, D)
        vwf = val_w[i].reshape(NE, D, D)
        for l in range(NL):
            ka, kb, va, vb = _proj(feat, kwf, vwf)
            agg_a, den_a = _edge_phase(ka.reshape(NE * N, HW),
                                       va.reshape(NE * N, HW),
                                       q_a, nkey, src, dst, et)
            agg_b, den_b = _edge_phase(kb.reshape(NE * N, HW),
                                       vb.reshape(NE * N, HW),
                                       q_b, nkey, src, dst, et)
            if l == NL - 1:
                feat, sc = _post_end(agg_a, agg_b, den_a, den_b, nk2d,
                                     wcat_a, wcat_b, prev, twf)
                prev = feat
                scores.append(sc[0, 0])
            else:
                feat = _post_mid(agg_a, agg_b, den_a, den_b, nk2d,
                                 wcat_a, wcat_b)
    return ((scores[0] + scores[1]) * 0.5).reshape(1)
